# baseline (speedup 1.0000x reference)
import functools

import jax
import jax.numpy as jnp
from jax.experimental import pallas as pl
from jax.experimental.pallas import tpu as pltpu

C = 16
K = 10
ALPHA = 0.9


def _lp_kernel(a_ref, yoh_ref, m_ref, out_ref, buf_ref, *, bi, ni):
    k = pl.program_id(0)
    i = pl.program_id(1)
    cur = jax.lax.rem(k, 2)

    base = m_ref[...] * yoh_ref[...]
    src = jnp.where(k == 0, base, buf_ref[cur])
    z = jnp.dot(a_ref[...], src, preferred_element_type=jnp.float32)

    sl = pl.ds(i * bi, bi)
    m_i = m_ref[sl, :]
    yoh_i = yoh_ref[sl, :]
    old_i = jnp.where(k == 0, m_i * yoh_i, buf_ref[cur, sl, :])
    val = jnp.clip(ALPHA * z + (1.0 - ALPHA) * old_i, 0.0, 1.0)
    new = jnp.where(m_i > 0.0, yoh_i, val)
    buf_ref[1 - cur, sl, :] = new

    @pl.when(k == K - 1)
    def _():
        out_ref[...] = new


def kernel(homo_adj, y, train_mask):
    n = homo_adj.shape[0]
    y_oh = jax.nn.one_hot(y.astype(jnp.int32), C, dtype=jnp.float32)
    maskf = jnp.broadcast_to(
        train_mask.astype(jnp.float32)[:, None], (n, C))

    bi = 400 if n % 400 == 0 else max(d for d in (8, 16, 32) if n % d == 0)
    ni = n // bi

    grid = (K, ni)
    body = functools.partial(_lp_kernel, bi=bi, ni=ni)
    out = pl.pallas_call(
        body,
        grid=grid,
        in_specs=[
            pl.BlockSpec((bi, n), lambda k, i: (i, 0)),
            pl.BlockSpec((n, C), lambda k, i: (0, 0)),
            pl.BlockSpec((n, C), lambda k, i: (0, 0)),
        ],
        out_specs=pl.BlockSpec(
            (bi, C),
            lambda k, i: (jax.lax.select(k == K - 1, i, ni), 0)),
        out_shape=jax.ShapeDtypeStruct(((ni + 1) * bi, C), jnp.float32),
        scratch_shapes=[
            pltpu.VMEM((2, n, C), jnp.float32),
        ],
    )(homo_adj, y_oh, maskf)
    return out[:n]


# speedup vs baseline: 1.2787x; 1.2787x over previous
import functools

import jax
import jax.numpy as jnp
from jax.experimental import pallas as pl
from jax.experimental.pallas import tpu as pltpu

C = 16
K = 10
ALPHA = 0.9


def _lp_kernel(a_ref, yoh_ref, m_ref, out_ref, buf_ref, *, bi, ni):
    k = pl.program_id(0)
    i = pl.program_id(1)
    cur = jax.lax.rem(k, 2)

    base = m_ref[...] * yoh_ref[...]
    src = jnp.where(k == 0, base, buf_ref[cur]).astype(jnp.bfloat16)
    z = jnp.dot(a_ref[...], src, preferred_element_type=jnp.float32)

    sl = pl.ds(i * bi, bi)
    m_i = m_ref[sl, :]
    yoh_i = yoh_ref[sl, :]
    old_i = jnp.where(k == 0, m_i * yoh_i, buf_ref[cur, sl, :])
    val = jnp.clip(ALPHA * z + (1.0 - ALPHA) * old_i, 0.0, 1.0)
    new = jnp.where(m_i > 0.0, yoh_i, val)
    buf_ref[1 - cur, sl, :] = new

    @pl.when(k == K - 1)
    def _():
        out_ref[...] = new


def kernel(homo_adj, y, train_mask):
    n = homo_adj.shape[0]
    a16 = homo_adj.astype(jnp.bfloat16)
    y_oh = jax.nn.one_hot(y.astype(jnp.int32), C, dtype=jnp.float32)
    maskf = jnp.broadcast_to(
        train_mask.astype(jnp.float32)[:, None], (n, C))

    bi = 400 if n % 400 == 0 else max(d for d in (8, 16, 32) if n % d == 0)
    ni = n // bi

    grid = (K, ni)
    body = functools.partial(_lp_kernel, bi=bi, ni=ni)
    out = pl.pallas_call(
        body,
        grid=grid,
        in_specs=[
            pl.BlockSpec((bi, n), lambda k, i: (i, 0)),
            pl.BlockSpec((n, C), lambda k, i: (0, 0)),
            pl.BlockSpec((n, C), lambda k, i: (0, 0)),
        ],
        out_specs=pl.BlockSpec(
            (bi, C),
            lambda k, i: (jax.lax.select(k == K - 1, i, ni), 0)),
        out_shape=jax.ShapeDtypeStruct(((ni + 1) * bi, C), jnp.float32),
        scratch_shapes=[
            pltpu.VMEM((2, n, C), jnp.float32),
        ],
    )(a16, y_oh, maskf)
    return out[:n]
